# baseline (speedup 1.0000x reference)
import functools

import jax
import jax.numpy as jnp
from jax import lax
from jax.experimental import pallas as pl
from jax.experimental.pallas import tpu as pltpu
from jax.experimental.pallas import tpu_sc as plsc

_N = 10000
_E = 320000
_DIN = 128
_DH = 16
_DOUT = 60

_NW = 32
_BA = 128
_KB = 80
_EPW = _KB * _BA
_NPAD = 10112
_RPS = _NPAD // 16

_RB = 10000


def _segsum(src_g, dst_g, y):
    mesh = plsc.VectorSubcoreMesh(core_axis_name="c", subcore_axis_name="s")
    zrows = jnp.zeros((_RPS, 16), jnp.float32)

    @functools.partial(
        pl.kernel,
        mesh=mesh,
        out_type=jax.ShapeDtypeStruct((2, _NPAD, 16), jnp.float32),
        compiler_params=pltpu.CompilerParams(use_tc_tiling_on_sc=False),
        scratch_types=[
            pltpu.VMEM((_KB, _BA), jnp.int32),
            pltpu.VMEM((_KB, _BA), jnp.int32),
            pltpu.VMEM((8, _BA, 16), jnp.float32),
            pltpu.VMEM_SHARED((_NPAD, 16), jnp.float32),
            pltpu.SemaphoreType.DMA((8,)),
            pltpu.SemaphoreType.DMA((8,)),
        ],
    )
    def run(zr_hbm, src_hbm, dst_hbm, y_hbm, out_hbm,
            src_v, dst_v, bufs, acc, gsems, ssems):
        c = lax.axis_index("c")
        s = lax.axis_index("s")
        wid = c * 16 + s
        pltpu.sync_copy(zr_hbm, acc.at[pl.ds(s * _RPS, _RPS)])
        pltpu.sync_copy(src_hbm.at[wid], src_v)
        pltpu.sync_copy(dst_hbm.at[wid], dst_v)
        plsc.subcore_barrier()

        for b in range(4):
            pltpu.async_copy(y_hbm.at[src_v.at[b]], bufs.at[b],
                             gsems.at[b])

        def body(G, carry):
            for b in range(8):
                j = 8 * G + b
                pltpu.make_async_copy(y_hbm.at[src_v.at[j]], bufs.at[b],
                                      gsems.at[b]).wait()
                pltpu.async_copy(bufs.at[b], acc.at[dst_v.at[j]],
                                 ssems.at[b], add=True)
                b4 = (b + 4) % 8

                @pl.when(j >= 4)
                def _():
                    pltpu.make_async_copy(bufs.at[b4],
                                          acc.at[dst_v.at[0]],
                                          ssems.at[b4]).wait()

                @pl.when(j + 4 < _KB)
                def _():
                    pltpu.async_copy(y_hbm.at[src_v.at[j + 4]],
                                     bufs.at[b4], gsems.at[b4])

            return carry

        lax.fori_loop(0, _KB // 8, body, 0)
        for b in range(4, 8):
            pltpu.make_async_copy(bufs.at[b], acc.at[dst_v.at[0]],
                                  ssems.at[b]).wait()
        plsc.subcore_barrier()
        pltpu.sync_copy(acc.at[pl.ds(s * _RPS, _RPS)],
                        out_hbm.at[c, pl.ds(s * _RPS, _RPS)])

    return run(zrows, src_g, dst_g, y)


def _tc1(x, wr_t, wl_t, b):
    def body(x_ref, wr_ref, wl_ref, b_ref, root_ref, rel_ref):
        xb = x_ref[...]
        root_ref[...] = (
            jnp.dot(xb, wr_ref[...], preferred_element_type=jnp.float32)
            + b_ref[...])
        rel_ref[...] = jnp.dot(xb, wl_ref[...],
                               preferred_element_type=jnp.float32)

    return pl.pallas_call(
        body,
        grid=(_N // _RB,),
        in_specs=[
            pl.BlockSpec((_RB, _DIN), lambda i: (i, 0)),
            pl.BlockSpec((_DIN, _DH), lambda i: (0, 0)),
            pl.BlockSpec((_DIN, _DH), lambda i: (0, 0)),
            pl.BlockSpec((1, _DH), lambda i: (0, 0)),
        ],
        out_specs=[
            pl.BlockSpec((_RB, _DH), lambda i: (i, 0)),
            pl.BlockSpec((_RB, _DH), lambda i: (i, 0)),
        ],
        out_shape=[jax.ShapeDtypeStruct((_N, _DH), jnp.float32)] * 2,
    )(x, wr_t, wl_t, b.reshape(1, _DH))


def _tc2(root_in, a0, a1, wr_t, b):
    def body(r_ref, a0_ref, a1_ref, wr_ref, b_ref, h_ref, root_ref):
        h = jnp.tanh(r_ref[...] + a0_ref[...] + a1_ref[...])
        h_ref[...] = h
        root_ref[...] = (
            jnp.dot(h, wr_ref[...], preferred_element_type=jnp.float32)
            + b_ref[...])

    return pl.pallas_call(
        body,
        grid=(_N // _RB,),
        in_specs=[
            pl.BlockSpec((_RB, _DH), lambda i: (i, 0)),
            pl.BlockSpec((_RB, _DH), lambda i: (i, 0)),
            pl.BlockSpec((_RB, _DH), lambda i: (i, 0)),
            pl.BlockSpec((_DH, _DH), lambda i: (0, 0)),
            pl.BlockSpec((1, _DH), lambda i: (0, 0)),
        ],
        out_specs=[
            pl.BlockSpec((_RB, _DH), lambda i: (i, 0)),
            pl.BlockSpec((_RB, _DH), lambda i: (i, 0)),
        ],
        out_shape=[jax.ShapeDtypeStruct((_N, _DH), jnp.float32)] * 2,
    )(root_in, a0, a1, wr_t, b.reshape(1, _DH))


def _tc3(root_in, a0, a1, wl2_t, wr3_t, b):
    def body(r_ref, a0_ref, a1_ref, wl_ref, wr_ref, b_ref,
             h_ref, root_ref):
        agg = a0_ref[...] + a1_ref[...]
        h = jnp.tanh(r_ref[...] + jnp.dot(
            agg, wl_ref[...], preferred_element_type=jnp.float32))
        h_ref[...] = h
        root_ref[...] = (
            jnp.dot(h, wr_ref[...], preferred_element_type=jnp.float32)
            + b_ref[...])

    return pl.pallas_call(
        body,
        grid=(_N // _RB,),
        in_specs=[
            pl.BlockSpec((_RB, _DH), lambda i: (i, 0)),
            pl.BlockSpec((_RB, _DH), lambda i: (i, 0)),
            pl.BlockSpec((_RB, _DH), lambda i: (i, 0)),
            pl.BlockSpec((_DH, _DH), lambda i: (0, 0)),
            pl.BlockSpec((_DH, _DOUT), lambda i: (0, 0)),
            pl.BlockSpec((1, _DOUT), lambda i: (0, 0)),
        ],
        out_specs=[
            pl.BlockSpec((_RB, _DH), lambda i: (i, 0)),
            pl.BlockSpec((_RB, _DOUT), lambda i: (i, 0)),
        ],
        out_shape=[
            jax.ShapeDtypeStruct((_N, _DH), jnp.float32),
            jax.ShapeDtypeStruct((_N, _DOUT), jnp.float32),
        ],
    )(root_in, a0, a1, wl2_t, wr3_t, b.reshape(1, _DOUT))


def _tc4(root3, a0, a1, wl_t):
    def body(r_ref, a0_ref, a1_ref, wl_ref, out_ref):
        agg = a0_ref[...] + a1_ref[...]
        out_ref[...] = r_ref[...] + jnp.dot(
            agg, wl_ref[...], preferred_element_type=jnp.float32)

    return pl.pallas_call(
        body,
        grid=(_N // _RB,),
        in_specs=[
            pl.BlockSpec((_RB, _DOUT), lambda i: (i, 0)),
            pl.BlockSpec((_RB, _DH), lambda i: (i, 0)),
            pl.BlockSpec((_RB, _DH), lambda i: (i, 0)),
            pl.BlockSpec((_DH, _DOUT), lambda i: (0, 0)),
        ],
        out_specs=pl.BlockSpec((_RB, _DOUT), lambda i: (i, 0)),
        out_shape=jax.ShapeDtypeStruct((_N, _DOUT), jnp.float32),
    )(root3, a0, a1, wl_t)


def kernel(x, edge_index, W1_root, W1_rel, b1, W2_root, W2_rel, b2,
           W3_root, W3_rel, b3):
    src = edge_index[0]
    dst = edge_index[1]
    pad = _NW * _EPW - _E
    pad_i = jnp.arange(pad, dtype=jnp.int32)
    src_g = jnp.concatenate(
        [src, pad_i % _N]).reshape(_NW, _KB, _BA)
    dst_g = jnp.concatenate(
        [dst, _N + pad_i % (_NPAD - _N)]).reshape(_NW, _KB, _BA)

    root1, rel1 = _tc1(x, W1_root.T, W1_rel.T, b1)
    a = _segsum(src_g, dst_g, rel1)
    h1, root2 = _tc2(root1, a[0, :_N], a[1, :_N], W2_root.T, b2)
    a = _segsum(src_g, dst_g, h1)
    h2, root3 = _tc3(root2, a[0, :_N], a[1, :_N], W2_rel.T, W3_root.T, b3)
    a = _segsum(src_g, dst_g, h2)
    return _tc4(root3, a[0, :_N], a[1, :_N], W3_rel.T)


# speedup vs baseline: 1.0891x; 1.0891x over previous
import functools

import jax
import jax.numpy as jnp
from jax import lax
from jax.experimental import pallas as pl
from jax.experimental.pallas import tpu as pltpu
from jax.experimental.pallas import tpu_sc as plsc

_N = 10000
_E = 320000
_DIN = 128
_DH = 16
_DOUT = 60

_NW = 32
_BA = 128
_KB = 80
_EPW = _KB * _BA
_NPAD = 10112
_RPS = _NPAD // 16

_RB = 10000


def _segsum(src_g, dst_g, y):
    mesh = plsc.VectorSubcoreMesh(core_axis_name="c", subcore_axis_name="s")
    zrows = jnp.zeros((_RPS, 16), jnp.float32)

    @functools.partial(
        pl.kernel,
        mesh=mesh,
        out_type=jax.ShapeDtypeStruct((2, _NPAD, 16), jnp.float32),
        compiler_params=pltpu.CompilerParams(use_tc_tiling_on_sc=False),
        scratch_types=[
            pltpu.VMEM((_KB, _BA), jnp.int32),
            pltpu.VMEM((_KB, _BA), jnp.int32),
            pltpu.VMEM((8, _BA, 16), jnp.float32),
            pltpu.VMEM_SHARED((_NPAD, 16), jnp.float32),
            pltpu.VMEM_SHARED((_N, 16), jnp.float32),
            pltpu.SemaphoreType.DMA((8,)),
            pltpu.SemaphoreType.DMA((8,)),
        ],
    )
    def run(zr_hbm, src_hbm, dst_hbm, y_hbm, out_hbm,
            src_v, dst_v, bufs, acc, ysh, gsems, ssems):
        c = lax.axis_index("c")
        s = lax.axis_index("s")
        wid = c * 16 + s
        pltpu.sync_copy(zr_hbm, acc.at[pl.ds(s * _RPS, _RPS)])
        pltpu.sync_copy(src_hbm.at[wid], src_v)
        pltpu.sync_copy(dst_hbm.at[wid], dst_v)

        @pl.when(s < 15)
        def _():
            pltpu.sync_copy(y_hbm.at[pl.ds(s * _RPS, _RPS)],
                            ysh.at[pl.ds(s * _RPS, _RPS)])

        @pl.when(s == 15)
        def _():
            pltpu.sync_copy(y_hbm.at[pl.ds(15 * _RPS, _N - 15 * _RPS)],
                            ysh.at[pl.ds(15 * _RPS, _N - 15 * _RPS)])

        plsc.subcore_barrier()

        for b in range(4):
            pltpu.async_copy(ysh.at[src_v.at[b]], bufs.at[b],
                             gsems.at[b])

        def body(G, carry):
            for b in range(8):
                j = 8 * G + b
                pltpu.make_async_copy(ysh.at[src_v.at[j]], bufs.at[b],
                                      gsems.at[b]).wait()
                pltpu.async_copy(bufs.at[b], acc.at[dst_v.at[j]],
                                 ssems.at[b], add=True)
                b4 = (b + 4) % 8

                @pl.when(j >= 4)
                def _():
                    pltpu.make_async_copy(bufs.at[b4],
                                          acc.at[dst_v.at[0]],
                                          ssems.at[b4]).wait()

                @pl.when(j + 4 < _KB)
                def _():
                    pltpu.async_copy(ysh.at[src_v.at[j + 4]],
                                     bufs.at[b4], gsems.at[b4])

            return carry

        lax.fori_loop(0, _KB // 8, body, 0)
        for b in range(4, 8):
            pltpu.make_async_copy(bufs.at[b], acc.at[dst_v.at[0]],
                                  ssems.at[b]).wait()
        plsc.subcore_barrier()
        pltpu.sync_copy(acc.at[pl.ds(s * _RPS, _RPS)],
                        out_hbm.at[c, pl.ds(s * _RPS, _RPS)])

    return run(zrows, src_g, dst_g, y)


def _tc1(x, wr_t, wl_t, b):
    def body(x_ref, wr_ref, wl_ref, b_ref, root_ref, rel_ref):
        xb = x_ref[...]
        root_ref[...] = (
            jnp.dot(xb, wr_ref[...], preferred_element_type=jnp.float32)
            + b_ref[...])
        rel_ref[...] = jnp.dot(xb, wl_ref[...],
                               preferred_element_type=jnp.float32)

    return pl.pallas_call(
        body,
        grid=(_N // _RB,),
        in_specs=[
            pl.BlockSpec((_RB, _DIN), lambda i: (i, 0)),
            pl.BlockSpec((_DIN, _DH), lambda i: (0, 0)),
            pl.BlockSpec((_DIN, _DH), lambda i: (0, 0)),
            pl.BlockSpec((1, _DH), lambda i: (0, 0)),
        ],
        out_specs=[
            pl.BlockSpec((_RB, _DH), lambda i: (i, 0)),
            pl.BlockSpec((_RB, _DH), lambda i: (i, 0)),
        ],
        out_shape=[jax.ShapeDtypeStruct((_N, _DH), jnp.float32)] * 2,
    )(x, wr_t, wl_t, b.reshape(1, _DH))


def _tc2(root_in, a0, a1, wr_t, b):
    def body(r_ref, a0_ref, a1_ref, wr_ref, b_ref, h_ref, root_ref):
        h = jnp.tanh(r_ref[...] + a0_ref[...] + a1_ref[...])
        h_ref[...] = h
        root_ref[...] = (
            jnp.dot(h, wr_ref[...], preferred_element_type=jnp.float32)
            + b_ref[...])

    return pl.pallas_call(
        body,
        grid=(_N // _RB,),
        in_specs=[
            pl.BlockSpec((_RB, _DH), lambda i: (i, 0)),
            pl.BlockSpec((_RB, _DH), lambda i: (i, 0)),
            pl.BlockSpec((_RB, _DH), lambda i: (i, 0)),
            pl.BlockSpec((_DH, _DH), lambda i: (0, 0)),
            pl.BlockSpec((1, _DH), lambda i: (0, 0)),
        ],
        out_specs=[
            pl.BlockSpec((_RB, _DH), lambda i: (i, 0)),
            pl.BlockSpec((_RB, _DH), lambda i: (i, 0)),
        ],
        out_shape=[jax.ShapeDtypeStruct((_N, _DH), jnp.float32)] * 2,
    )(root_in, a0, a1, wr_t, b.reshape(1, _DH))


def _tc3(root_in, a0, a1, wl2_t, wr3_t, b):
    def body(r_ref, a0_ref, a1_ref, wl_ref, wr_ref, b_ref,
             h_ref, root_ref):
        agg = a0_ref[...] + a1_ref[...]
        h = jnp.tanh(r_ref[...] + jnp.dot(
            agg, wl_ref[...], preferred_element_type=jnp.float32))
        h_ref[...] = h
        root_ref[...] = (
            jnp.dot(h, wr_ref[...], preferred_element_type=jnp.float32)
            + b_ref[...])

    return pl.pallas_call(
        body,
        grid=(_N // _RB,),
        in_specs=[
            pl.BlockSpec((_RB, _DH), lambda i: (i, 0)),
            pl.BlockSpec((_RB, _DH), lambda i: (i, 0)),
            pl.BlockSpec((_RB, _DH), lambda i: (i, 0)),
            pl.BlockSpec((_DH, _DH), lambda i: (0, 0)),
            pl.BlockSpec((_DH, _DOUT), lambda i: (0, 0)),
            pl.BlockSpec((1, _DOUT), lambda i: (0, 0)),
        ],
        out_specs=[
            pl.BlockSpec((_RB, _DH), lambda i: (i, 0)),
            pl.BlockSpec((_RB, _DOUT), lambda i: (i, 0)),
        ],
        out_shape=[
            jax.ShapeDtypeStruct((_N, _DH), jnp.float32),
            jax.ShapeDtypeStruct((_N, _DOUT), jnp.float32),
        ],
    )(root_in, a0, a1, wl2_t, wr3_t, b.reshape(1, _DOUT))


def _tc4(root3, a0, a1, wl_t):
    def body(r_ref, a0_ref, a1_ref, wl_ref, out_ref):
        agg = a0_ref[...] + a1_ref[...]
        out_ref[...] = r_ref[...] + jnp.dot(
            agg, wl_ref[...], preferred_element_type=jnp.float32)

    return pl.pallas_call(
        body,
        grid=(_N // _RB,),
        in_specs=[
            pl.BlockSpec((_RB, _DOUT), lambda i: (i, 0)),
            pl.BlockSpec((_RB, _DH), lambda i: (i, 0)),
            pl.BlockSpec((_RB, _DH), lambda i: (i, 0)),
            pl.BlockSpec((_DH, _DOUT), lambda i: (0, 0)),
        ],
        out_specs=pl.BlockSpec((_RB, _DOUT), lambda i: (i, 0)),
        out_shape=jax.ShapeDtypeStruct((_N, _DOUT), jnp.float32),
    )(root3, a0, a1, wl_t)


def kernel(x, edge_index, W1_root, W1_rel, b1, W2_root, W2_rel, b2,
           W3_root, W3_rel, b3):
    src = edge_index[0]
    dst = edge_index[1]
    pad = _NW * _EPW - _E
    pad_i = jnp.arange(pad, dtype=jnp.int32)
    src_g = jnp.concatenate(
        [src, pad_i % _N]).reshape(_NW, _KB, _BA)
    dst_g = jnp.concatenate(
        [dst, _N + pad_i % (_NPAD - _N)]).reshape(_NW, _KB, _BA)

    root1, rel1 = _tc1(x, W1_root.T, W1_rel.T, b1)
    a = _segsum(src_g, dst_g, rel1)
    h1, root2 = _tc2(root1, a[0, :_N], a[1, :_N], W2_root.T, b2)
    a = _segsum(src_g, dst_g, h1)
    h2, root3 = _tc3(root2, a[0, :_N], a[1, :_N], W2_rel.T, W3_root.T, b3)
    a = _segsum(src_g, dst_g, h2)
    return _tc4(root3, a[0, :_N], a[1, :_N], W3_rel.T)


# speedup vs baseline: 1.1285x; 1.0362x over previous
import functools

import jax
import jax.numpy as jnp
from jax import lax
from jax.experimental import pallas as pl
from jax.experimental.pallas import tpu as pltpu
from jax.experimental.pallas import tpu_sc as plsc

_N = 10000
_E = 320000
_DIN = 128
_DH = 16
_DOUT = 60

_NW = 32
_BA = 125
_KB = 80
_NPAD = 10112
_RPS = _NPAD // 16

_RB = 10000


def _segsum(ei_g, y):
    mesh = plsc.VectorSubcoreMesh(core_axis_name="c", subcore_axis_name="s")
    zrows = jnp.zeros((_RPS, 16), jnp.float32)

    @functools.partial(
        pl.kernel,
        mesh=mesh,
        out_type=jax.ShapeDtypeStruct((2, _NPAD, 16), jnp.float32),
        compiler_params=pltpu.CompilerParams(use_tc_tiling_on_sc=False),
        scratch_types=[
            pltpu.VMEM((_KB, _BA), jnp.int32),
            pltpu.VMEM((_KB, _BA), jnp.int32),
            pltpu.VMEM((8, _BA, 16), jnp.float32),
            pltpu.VMEM_SHARED((_NPAD, 16), jnp.float32),
            pltpu.VMEM_SHARED((_N, 16), jnp.float32),
            pltpu.SemaphoreType.DMA((8,)),
            pltpu.SemaphoreType.DMA((8,)),
        ],
    )
    def run(zr_hbm, ei_hbm, y_hbm, out_hbm,
            src_v, dst_v, bufs, acc, ysh, gsems, ssems):
        c = lax.axis_index("c")
        s = lax.axis_index("s")
        wid = c * 16 + s
        pltpu.sync_copy(zr_hbm, acc.at[pl.ds(s * _RPS, _RPS)])
        pltpu.sync_copy(ei_hbm.at[0, wid], src_v)
        pltpu.sync_copy(ei_hbm.at[1, wid], dst_v)

        @pl.when(s < 15)
        def _():
            pltpu.sync_copy(y_hbm.at[pl.ds(s * _RPS, _RPS)],
                            ysh.at[pl.ds(s * _RPS, _RPS)])

        @pl.when(s == 15)
        def _():
            pltpu.sync_copy(y_hbm.at[pl.ds(15 * _RPS, _N - 15 * _RPS)],
                            ysh.at[pl.ds(15 * _RPS, _N - 15 * _RPS)])

        plsc.subcore_barrier()

        for b in range(4):
            pltpu.async_copy(ysh.at[src_v.at[b]], bufs.at[b],
                             gsems.at[b])

        def body(G, carry):
            for b in range(8):
                j = 8 * G + b
                pltpu.make_async_copy(ysh.at[src_v.at[j]], bufs.at[b],
                                      gsems.at[b]).wait()
                pltpu.async_copy(bufs.at[b], acc.at[dst_v.at[j]],
                                 ssems.at[b], add=True)
                b4 = (b + 4) % 8

                @pl.when(j >= 4)
                def _():
                    pltpu.make_async_copy(bufs.at[b4],
                                          acc.at[dst_v.at[0]],
                                          ssems.at[b4]).wait()

                @pl.when(j + 4 < _KB)
                def _():
                    pltpu.async_copy(ysh.at[src_v.at[j + 4]],
                                     bufs.at[b4], gsems.at[b4])

            return carry

        lax.fori_loop(0, _KB // 8, body, 0)
        for b in range(4, 8):
            pltpu.make_async_copy(bufs.at[b], acc.at[dst_v.at[0]],
                                  ssems.at[b]).wait()
        plsc.subcore_barrier()
        pltpu.sync_copy(acc.at[pl.ds(s * _RPS, _RPS)],
                        out_hbm.at[c, pl.ds(s * _RPS, _RPS)])

    return run(zrows, ei_g, y)


def _tc1(x, wr_t, wl_t, b):
    def body(x_ref, wr_ref, wl_ref, b_ref, root_ref, rel_ref):
        xb = x_ref[...]
        root_ref[...] = (
            jnp.dot(xb, wr_ref[...], preferred_element_type=jnp.float32)
            + b_ref[...])
        rel_ref[...] = jnp.dot(xb, wl_ref[...],
                               preferred_element_type=jnp.float32)

    return pl.pallas_call(
        body,
        grid=(_N // _RB,),
        in_specs=[
            pl.BlockSpec((_RB, _DIN), lambda i: (i, 0)),
            pl.BlockSpec((_DIN, _DH), lambda i: (0, 0)),
            pl.BlockSpec((_DIN, _DH), lambda i: (0, 0)),
            pl.BlockSpec((1, _DH), lambda i: (0, 0)),
        ],
        out_specs=[
            pl.BlockSpec((_RB, _DH), lambda i: (i, 0)),
            pl.BlockSpec((_RB, _DH), lambda i: (i, 0)),
        ],
        out_shape=[jax.ShapeDtypeStruct((_N, _DH), jnp.float32)] * 2,
    )(x, wr_t, wl_t, b.reshape(1, _DH))


def _tc2(root_in, a0, a1, wr_t, b):
    def body(r_ref, a0_ref, a1_ref, wr_ref, b_ref, h_ref, root_ref):
        h = jnp.tanh(r_ref[...] + a0_ref[...] + a1_ref[...])
        h_ref[...] = h
        root_ref[...] = (
            jnp.dot(h, wr_ref[...], preferred_element_type=jnp.float32)
            + b_ref[...])

    return pl.pallas_call(
        body,
        grid=(_N // _RB,),
        in_specs=[
            pl.BlockSpec((_RB, _DH), lambda i: (i, 0)),
            pl.BlockSpec((_RB, _DH), lambda i: (i, 0)),
            pl.BlockSpec((_RB, _DH), lambda i: (i, 0)),
            pl.BlockSpec((_DH, _DH), lambda i: (0, 0)),
            pl.BlockSpec((1, _DH), lambda i: (0, 0)),
        ],
        out_specs=[
            pl.BlockSpec((_RB, _DH), lambda i: (i, 0)),
            pl.BlockSpec((_RB, _DH), lambda i: (i, 0)),
        ],
        out_shape=[jax.ShapeDtypeStruct((_N, _DH), jnp.float32)] * 2,
    )(root_in, a0, a1, wr_t, b.reshape(1, _DH))


def _tc3(root_in, a0, a1, wl2_t, wr3_t, b):
    def body(r_ref, a0_ref, a1_ref, wl_ref, wr_ref, b_ref,
             h_ref, root_ref):
        agg = a0_ref[...] + a1_ref[...]
        h = jnp.tanh(r_ref[...] + jnp.dot(
            agg, wl_ref[...], preferred_element_type=jnp.float32))
        h_ref[...] = h
        root_ref[...] = (
            jnp.dot(h, wr_ref[...], preferred_element_type=jnp.float32)
            + b_ref[...])

    return pl.pallas_call(
        body,
        grid=(_N // _RB,),
        in_specs=[
            pl.BlockSpec((_RB, _DH), lambda i: (i, 0)),
            pl.BlockSpec((_RB, _DH), lambda i: (i, 0)),
            pl.BlockSpec((_RB, _DH), lambda i: (i, 0)),
            pl.BlockSpec((_DH, _DH), lambda i: (0, 0)),
            pl.BlockSpec((_DH, _DOUT), lambda i: (0, 0)),
            pl.BlockSpec((1, _DOUT), lambda i: (0, 0)),
        ],
        out_specs=[
            pl.BlockSpec((_RB, _DH), lambda i: (i, 0)),
            pl.BlockSpec((_RB, _DOUT), lambda i: (i, 0)),
        ],
        out_shape=[
            jax.ShapeDtypeStruct((_N, _DH), jnp.float32),
            jax.ShapeDtypeStruct((_N, _DOUT), jnp.float32),
        ],
    )(root_in, a0, a1, wl2_t, wr3_t, b.reshape(1, _DOUT))


def _tc4(root3, a0, a1, wl_t):
    def body(r_ref, a0_ref, a1_ref, wl_ref, out_ref):
        agg = a0_ref[...] + a1_ref[...]
        out_ref[...] = r_ref[...] + jnp.dot(
            agg, wl_ref[...], preferred_element_type=jnp.float32)

    return pl.pallas_call(
        body,
        grid=(_N // _RB,),
        in_specs=[
            pl.BlockSpec((_RB, _DOUT), lambda i: (i, 0)),
            pl.BlockSpec((_RB, _DH), lambda i: (i, 0)),
            pl.BlockSpec((_RB, _DH), lambda i: (i, 0)),
            pl.BlockSpec((_DH, _DOUT), lambda i: (0, 0)),
        ],
        out_specs=pl.BlockSpec((_RB, _DOUT), lambda i: (i, 0)),
        out_shape=jax.ShapeDtypeStruct((_N, _DOUT), jnp.float32),
    )(root3, a0, a1, wl_t)


def kernel(x, edge_index, W1_root, W1_rel, b1, W2_root, W2_rel, b2,
           W3_root, W3_rel, b3):
    ei_g = edge_index.reshape(2, _NW, _KB, _BA)

    root1, rel1 = _tc1(x, W1_root.T, W1_rel.T, b1)
    a = _segsum(ei_g, rel1)
    h1, root2 = _tc2(root1, a[0, :_N], a[1, :_N], W2_root.T, b2)
    a = _segsum(ei_g, h1)
    h2, root3 = _tc3(root2, a[0, :_N], a[1, :_N], W2_rel.T, W3_root.T, b3)
    a = _segsum(ei_g, h2)
    return _tc4(root3, a[0, :_N], a[1, :_N], W3_rel.T)


# speedup vs baseline: 1.8423x; 1.6325x over previous
import functools

import jax
import jax.numpy as jnp
from jax import lax
from jax.experimental import pallas as pl
from jax.experimental.pallas import tpu as pltpu
from jax.experimental.pallas import tpu_sc as plsc

_N = 10000
_E = 320000
_DIN = 128
_DH = 16
_DOUT = 60

_NW = 32
_BA = 125
_KB = 80
_NPAD = 10112
_RPS = _NPAD // 16

_P = _N // 8
_PA = _NPAD // 8


def _segsum(ei_g, y):
    mesh = plsc.VectorSubcoreMesh(core_axis_name="c", subcore_axis_name="s")
    zrows = jnp.zeros((_RPS, 16), jnp.float32)

    @functools.partial(
        pl.kernel,
        mesh=mesh,
        out_type=jax.ShapeDtypeStruct((2, _NPAD, 16), jnp.float32),
        compiler_params=pltpu.CompilerParams(use_tc_tiling_on_sc=False),
        scratch_types=[
            pltpu.VMEM((_KB, _BA), jnp.int32),
            pltpu.VMEM((_KB, _BA), jnp.int32),
            pltpu.VMEM((8, _BA, 16), jnp.float32),
            pltpu.VMEM_SHARED((_NPAD, 16), jnp.float32),
            pltpu.VMEM_SHARED((_N, 16), jnp.float32),
            pltpu.SemaphoreType.DMA((8,)),
            pltpu.SemaphoreType.DMA((8,)),
        ],
    )
    def run(zr_hbm, ei_hbm, y_hbm, out_hbm,
            src_v, dst_v, bufs, acc, ysh, gsems, ssems):
        c = lax.axis_index("c")
        s = lax.axis_index("s")
        wid = c * 16 + s
        pltpu.sync_copy(zr_hbm, acc.at[pl.ds(s * _RPS, _RPS)])
        pltpu.sync_copy(ei_hbm.at[0, wid], src_v)
        pltpu.sync_copy(ei_hbm.at[1, wid], dst_v)

        @pl.when(s < 15)
        def _():
            pltpu.sync_copy(y_hbm.at[pl.ds(s * _RPS, _RPS)],
                            ysh.at[pl.ds(s * _RPS, _RPS)])

        @pl.when(s == 15)
        def _():
            pltpu.sync_copy(y_hbm.at[pl.ds(15 * _RPS, _N - 15 * _RPS)],
                            ysh.at[pl.ds(15 * _RPS, _N - 15 * _RPS)])

        plsc.subcore_barrier()

        for b in range(4):
            pltpu.async_copy(ysh.at[src_v.at[b]], bufs.at[b],
                             gsems.at[b])

        def body(G, carry):
            for b in range(8):
                j = 8 * G + b
                pltpu.make_async_copy(ysh.at[src_v.at[j]], bufs.at[b],
                                      gsems.at[b]).wait()
                pltpu.async_copy(bufs.at[b], acc.at[dst_v.at[j]],
                                 ssems.at[b], add=True)
                b4 = (b + 4) % 8

                @pl.when(j >= 4)
                def _():
                    pltpu.make_async_copy(bufs.at[b4],
                                          acc.at[dst_v.at[0]],
                                          ssems.at[b4]).wait()

                @pl.when(j + 4 < _KB)
                def _():
                    pltpu.async_copy(ysh.at[src_v.at[j + 4]],
                                     bufs.at[b4], gsems.at[b4])

            return carry

        lax.fori_loop(0, _KB // 8, body, 0)
        for b in range(4, 8):
            pltpu.make_async_copy(bufs.at[b], acc.at[dst_v.at[0]],
                                  ssems.at[b]).wait()
        plsc.subcore_barrier()
        pltpu.sync_copy(acc.at[pl.ds(s * _RPS, _RPS)],
                        out_hbm.at[c, pl.ds(s * _RPS, _RPS)])

    return run(zrows, ei_g, y)


def _blockdiag(w_t, kb, nb, reps):
    tiled = jnp.tile(w_t, (reps, reps))
    i = lax.broadcasted_iota(jnp.int32, (reps * kb, reps * nb), 0)
    j = lax.broadcasted_iota(jnp.int32, (reps * kb, reps * nb), 1)
    return jnp.where((i // kb) == (j // nb), tiled, 0.0)


def _tc1(xp, wr, wl, b):
    def body(x_ref, wr_ref, wl_ref, b_ref, root_ref, rel_ref):
        xb = x_ref[...]
        blk_r = _blockdiag(wr_ref[...].T, _DIN, _DH, 8)
        blk_l = _blockdiag(wl_ref[...].T, _DIN, _DH, 8)
        bt = jnp.tile(b_ref[...], (1, 8))
        root_ref[...] = jnp.dot(
            xb, blk_r, preferred_element_type=jnp.float32) + bt
        rel_ref[...] = jnp.dot(xb, blk_l, preferred_element_type=jnp.float32)

    return pl.pallas_call(
        body,
        out_shape=[jax.ShapeDtypeStruct((_P, 128), jnp.float32)] * 2,
    )(xp, wr, wl, b.reshape(1, _DH))


def _tc2(root_in, ap, wr, b):
    def body(r_ref, a_ref, wr_ref, b_ref, h_ref, root_ref):
        a = a_ref[...]
        h = jnp.tanh(r_ref[...] + a[0, :_P, :] + a[1, :_P, :])
        h_ref[...] = h
        blk = _blockdiag(wr_ref[...].T, _DH, _DH, 8)
        bt = jnp.tile(b_ref[...], (1, 8))
        root_ref[...] = jnp.dot(
            h, blk, preferred_element_type=jnp.float32) + bt

    return pl.pallas_call(
        body,
        out_shape=[jax.ShapeDtypeStruct((_P, 128), jnp.float32)] * 2,
    )(root_in, ap, wr, b.reshape(1, _DH))


def _tc3(root_in, ap, wl):
    def body(r_ref, a_ref, wl_ref, h_ref):
        a = a_ref[...]
        agg = a[0, :_P, :] + a[1, :_P, :]
        blk = _blockdiag(wl_ref[...].T, _DH, _DH, 8)
        h_ref[...] = jnp.tanh(r_ref[...] + jnp.dot(
            agg, blk, preferred_element_type=jnp.float32))

    return pl.pallas_call(
        body,
        out_shape=jax.ShapeDtypeStruct((_P, 128), jnp.float32),
    )(root_in, ap, wl)


def _tc4(h2p, ap, wr, wl, b):
    def body(h_ref, a_ref, wr_ref, wl_ref, b_ref, out_ref):
        a = a_ref[...]
        agg = a[0, :_P, :] + a[1, :_P, :]
        blk_r = _blockdiag(wr_ref[...].T, _DH, _DOUT, 8)
        blk_l = _blockdiag(wl_ref[...].T, _DH, _DOUT, 8)
        d1 = jnp.dot(h_ref[...], blk_r, preferred_element_type=jnp.float32)
        d2 = jnp.dot(agg, blk_l, preferred_element_type=jnp.float32)
        out_ref[...] = (d1 + d2) + jnp.tile(b_ref[...], (1, 8))

    return pl.pallas_call(
        body,
        out_shape=jax.ShapeDtypeStruct((_P, 8 * _DOUT), jnp.float32),
    )(h2p, ap, wr, wl, b.reshape(1, _DOUT))


def kernel(x, edge_index, W1_root, W1_rel, b1, W2_root, W2_rel, b2,
           W3_root, W3_rel, b3):
    ei_g = edge_index.reshape(2, _NW, _KB, _BA)
    xp = x.reshape(_P, 8 * _DIN)

    root1, rel1 = _tc1(xp, W1_root, W1_rel, b1)
    a = _segsum(ei_g, rel1.reshape(_N, _DH))
    h1, root2 = _tc2(root1, a.reshape(2, _PA, 128), W2_root, b2)
    a = _segsum(ei_g, h1.reshape(_N, _DH))
    h2 = _tc3(root2, a.reshape(2, _PA, 128), W2_rel)
    a = _segsum(ei_g, h2.reshape(_N, _DH))
    out_p = _tc4(h2, a.reshape(2, _PA, 128), W3_root, W3_rel, b3)
    return out_p.reshape(_N, _DOUT)
